# baseline (speedup 1.0000x reference)
import functools

import jax
import jax.numpy as jnp
from jax import lax
from jax.experimental import pallas as pl
from jax.experimental.pallas import tpu as pltpu
from jax.experimental.pallas import tpu_sc as plsc

H = 4100
W = 4100
C = 4
B = 262144

NUM_WORKERS = 32
PER_WORKER = B // NUM_WORKERS
CHUNK = 2048
NCHUNKS = PER_WORKER // CHUNK
L = 16
PERIOD = float(W - 4)
NROWS = H * W * C // 8
HALFW = W // 2
NB = CHUNK // 128


def _make_kernel():
  mesh = plsc.VectorSubcoreMesh(core_axis_name="c", subcore_axis_name="s")

  @functools.partial(
      pl.kernel,
      mesh=mesh,
      compiler_params=pltpu.CompilerParams(
          needs_layout_passes=False, use_tc_tiling_on_sc=False),
      out_type=jax.ShapeDtypeStruct((B * C,), jnp.float32),
      scratch_types=[
          pltpu.VMEM((CHUNK,), jnp.float32),
          pltpu.VMEM((CHUNK,), jnp.float32),
          pltpu.VMEM((CHUNK,), jnp.float32),
          pltpu.VMEM((CHUNK,), jnp.float32),
          pltpu.VMEM((CHUNK,), jnp.int32),
          pltpu.VMEM((4 * NB, 128), jnp.int32),
          pltpu.VMEM((4 * CHUNK, 8), jnp.float32),
          pltpu.VMEM((CHUNK * C,), jnp.float32),
          pltpu.SemaphoreType.DMA,
      ],
  )
  def bilerp(y_hbm, x_hbm, vis_hbm, out_hbm,
             y_v, x_v, dy_v, dx_v, p0_v, idx_v, rows_v, o_v, sem):
    wid = lax.axis_index("s") * 2 + lax.axis_index("c")
    base = wid * PER_WORKER

    lane = lax.iota(jnp.int32, L)
    sh2 = lax.shift_right_logical(lane, 2)
    ch = lax.bitwise_and(lane, 3)

    for k in range(NCHUNKS):
      cbase = base + k * CHUNK
      pltpu.sync_copy(y_hbm.at[pl.ds(cbase, CHUNK)], y_v)
      pltpu.sync_copy(x_hbm.at[pl.ds(cbase, CHUNK)], x_v)

      def pass1(j, _):
        for u in range(128 // L):
          o = j * 128 + u * L
          y = y_v[pl.ds(o, L)]
          x = x_v[pl.ds(o, L)]
          cy = jnp.where(y >= 1.0, y, y + PERIOD)
          cx = jnp.where(x >= 1.0, x, x + PERIOD)
          iy = cy.astype(jnp.int32)
          ix = cx.astype(jnp.int32)
          dy_v[pl.ds(o, L)] = cy - iy.astype(jnp.float32)
          dx_v[pl.ds(o, L)] = cx - ix.astype(jnp.float32)
          p0 = iy * W + ix
          p0_v[pl.ds(o, L)] = p0
          b_tl = lax.shift_right_logical(p0, 1)
          b_bl = lax.shift_right_logical(p0 + 1, 1)
          c0 = u * L
          idx_v[j, pl.ds(c0, L)] = b_tl
          idx_v[NB + j, pl.ds(c0, L)] = b_bl
          idx_v[2 * NB + j, pl.ds(c0, L)] = b_tl + HALFW
          idx_v[3 * NB + j, pl.ds(c0, L)] = b_bl + HALFW
        for b in range(4):
          r = b * NB + j
          pltpu.async_copy(
              vis_hbm.at[idx_v.at[r]],
              rows_v.at[pl.ds(r * 128, 128)], sem)
        return 0

      lax.fori_loop(0, NB, pass1, 0)

      def drain(r, _):
        pltpu.make_async_copy(
            vis_hbm.at[idx_v.at[r]],
            rows_v.at[pl.ds(r * 128, 128)], sem).wait()
        return 0

      lax.fori_loop(0, 4 * NB, drain, 0)

      def pass2(v, _):
        pt = v * 4 + sh2
        d0 = plsc.load_gather(dy_v, [pt])
        d1 = plsc.load_gather(dx_v, [pt])
        p0 = plsc.load_gather(p0_v, [pt])
        a = lax.shift_left(lax.bitwise_and(p0, 1), 2)
        o_lo = a + ch
        o_hi = (4 + ch) - a
        tl = plsc.load_gather(rows_v, [pt, o_lo])
        bl = plsc.load_gather(rows_v, [pt + CHUNK, o_hi])
        tr = plsc.load_gather(rows_v, [pt + 2 * CHUNK, o_lo])
        br = plsc.load_gather(rows_v, [pt + 3 * CHUNK, o_hi])
        mb = br + d0 * (bl - br)
        mt = tr + d0 * (tl - tr)
        o_v[pl.ds(v * L, L)] = mb + d1 * (mt - mb)
        return 0

      lax.fori_loop(0, CHUNK // 4, pass2, 0)

      pltpu.sync_copy(o_v, out_hbm.at[pl.ds(cbase * C, CHUNK * C)])

  return bilerp


_bilerp = _make_kernel()


@jax.jit
def kernel(coords, visible):
  ct = coords.T
  y = ct[0]
  x = ct[1]
  vis = visible.reshape(NROWS, 8)
  out = _bilerp(y, x, vis)
  return out.reshape(B, C)


# speedup vs baseline: 1.0089x; 1.0089x over previous
import functools

import jax
import jax.numpy as jnp
from jax import lax
from jax.experimental import pallas as pl
from jax.experimental.pallas import tpu as pltpu
from jax.experimental.pallas import tpu_sc as plsc

H = 4100
W = 4100
C = 4
B = 262144

NUM_WORKERS = 32
PER_WORKER = B // NUM_WORKERS
CHUNK = 2048
NCHUNKS = PER_WORKER // CHUNK
L = 16
PERIOD = float(W - 4)
NROWS = H * W * C // 8
HALFW = W // 2
NB = CHUNK // 128


def _make_kernel():
  mesh = plsc.VectorSubcoreMesh(core_axis_name="c", subcore_axis_name="s")

  @functools.partial(
      pl.kernel,
      mesh=mesh,
      compiler_params=pltpu.CompilerParams(
          needs_layout_passes=False, use_tc_tiling_on_sc=False),
      out_type=jax.ShapeDtypeStruct((C, B), jnp.float32),
      scratch_types=[
          pltpu.VMEM((CHUNK,), jnp.float32),
          pltpu.VMEM((CHUNK,), jnp.float32),
          pltpu.VMEM((CHUNK,), jnp.float32),
          pltpu.VMEM((CHUNK,), jnp.float32),
          pltpu.VMEM((CHUNK,), jnp.int32),
          pltpu.VMEM((4 * NB, 128), jnp.int32),
          pltpu.VMEM((4 * CHUNK, 8), jnp.float32),
          pltpu.VMEM((C, CHUNK), jnp.float32),
          pltpu.SemaphoreType.DMA,
      ],
  )
  def bilerp(y_hbm, x_hbm, vis_hbm, out_hbm,
             y_v, x_v, dy_v, dx_v, p0_v, idx_v, rows_v, o_v, sem):
    wid = lax.axis_index("s") * 2 + lax.axis_index("c")
    base = wid * PER_WORKER

    lane = lax.iota(jnp.int32, L)

    for k in range(NCHUNKS):
      cbase = base + k * CHUNK
      pltpu.sync_copy(y_hbm.at[pl.ds(cbase, CHUNK)], y_v)
      pltpu.sync_copy(x_hbm.at[pl.ds(cbase, CHUNK)], x_v)

      def pass1(j, _):
        for u in range(128 // L):
          o = j * 128 + u * L
          y = y_v[pl.ds(o, L)]
          x = x_v[pl.ds(o, L)]
          cy = jnp.where(y >= 1.0, y, y + PERIOD)
          cx = jnp.where(x >= 1.0, x, x + PERIOD)
          iy = cy.astype(jnp.int32)
          ix = cx.astype(jnp.int32)
          dy_v[pl.ds(o, L)] = cy - iy.astype(jnp.float32)
          dx_v[pl.ds(o, L)] = cx - ix.astype(jnp.float32)
          p0 = iy * W + ix
          p0_v[pl.ds(o, L)] = p0
          b_tl = lax.shift_right_logical(p0, 1)
          b_bl = lax.shift_right_logical(p0 + 1, 1)
          c0 = u * L
          idx_v[j, pl.ds(c0, L)] = b_tl
          idx_v[NB + j, pl.ds(c0, L)] = b_bl
          idx_v[2 * NB + j, pl.ds(c0, L)] = b_tl + HALFW
          idx_v[3 * NB + j, pl.ds(c0, L)] = b_bl + HALFW
        for b in range(4):
          r = b * NB + j
          pltpu.async_copy(
              vis_hbm.at[idx_v.at[r]],
              rows_v.at[pl.ds(r * 128, 128)], sem)
        return 0

      lax.fori_loop(0, NB, pass1, 0)

      def drain(r, _):
        pltpu.make_async_copy(
            vis_hbm.at[idx_v.at[r]],
            rows_v.at[pl.ds(r * 128, 128)], sem).wait()
        return 0

      lax.fori_loop(0, 4 * NB, drain, 0)

      def pass2(g, _):
        o = g * L
        pt = o + lane
        d0 = dy_v[pl.ds(o, L)]
        d1 = dx_v[pl.ds(o, L)]
        p0 = p0_v[pl.ds(o, L)]
        a = lax.shift_left(lax.bitwise_and(p0, 1), 2)
        na = 4 - a
        pt1 = pt + CHUNK
        pt2 = pt + 2 * CHUNK
        pt3 = pt + 3 * CHUNK
        for c in range(C):
          o_lo = a + c
          o_hi = na + c
          tl = plsc.load_gather(rows_v, [pt, o_lo])
          bl = plsc.load_gather(rows_v, [pt1, o_hi])
          tr = plsc.load_gather(rows_v, [pt2, o_lo])
          br = plsc.load_gather(rows_v, [pt3, o_hi])
          mb = br + d0 * (bl - br)
          mt = tr + d0 * (tl - tr)
          o_v[c, pl.ds(o, L)] = mb + d1 * (mt - mb)
        return 0

      lax.fori_loop(0, CHUNK // L, pass2, 0)

      for c in range(C):
        pltpu.sync_copy(o_v.at[c], out_hbm.at[c, pl.ds(cbase, CHUNK)])

  return bilerp


_bilerp = _make_kernel()


@jax.jit
def kernel(coords, visible):
  ct = coords.T
  y = ct[0]
  x = ct[1]
  t1 = lax.optimization_barrier(jnp.transpose(visible, (0, 2, 1)))
  vis = jnp.transpose(t1, (0, 2, 1)).reshape(NROWS, 8)
  out = _bilerp(y, x, vis)
  return out.T
